# baseline (speedup 1.0000x reference)
import functools

import jax
import jax.numpy as jnp
from jax import lax
from jax.experimental import pallas as pl
from jax.experimental.pallas import tpu as pltpu
from jax.experimental.pallas import tpu_sc as plsc

_N = 512
_D = 128
_K = 10
_T = 2 * _K + 1
_B = 2 * _N - 1
_NC = 2
_RPC = _N // _NC
_LO = _N - 11
_FILL = _LO
_SEEDL = _B
_SEEDR = _B + 8
_RB = 496


@functools.partial(
    pl.kernel,
    out_type=jax.ShapeDtypeStruct((_N, _N, _D), jnp.float32),
    mesh=plsc.ScalarSubcoreMesh(axis_name="c", num_cores=_NC),
    scratch_types=[
        pltpu.VMEM_SHARED((_B + 16, _D), jnp.float32),
        pltpu.SemaphoreType.DMA,
    ],
)
def _sc_band_fill(table_hbm, out_hbm, b_sh, sem):
    cid = lax.axis_index("c")
    r0 = cid * _RPC
    pltpu.sync_copy(table_hbm, b_sh.at[pl.ds(_LO, _T)])
    seeds = []
    for k in range(8):
        seeds.append(
            pltpu.async_copy(
                b_sh.at[pl.ds(_LO, 1)], b_sh.at[pl.ds(_SEEDL + k, 1)], sem
            )
        )
        seeds.append(
            pltpu.async_copy(
                b_sh.at[pl.ds(_LO + _T - 1, 1)],
                b_sh.at[pl.ds(_SEEDR + k, 1)],
                sem,
            )
        )
    for c in seeds:
        c.wait()
    lscr = out_hbm.at[r0]
    rscr = out_hbm.at[r0 + 1]
    s0 = pltpu.async_copy(b_sh.at[pl.ds(_SEEDL, 8)], lscr.at[pl.ds(0, 8)], sem)
    s1 = pltpu.async_copy(b_sh.at[pl.ds(_SEEDR, 8)], rscr.at[pl.ds(0, 8)], sem)
    s0.wait()
    s1.wait()
    for have in (8, 64):
        copies = []
        for k in range(1, 8):
            copies.append(
                pltpu.async_copy(
                    lscr.at[pl.ds(0, have)], lscr.at[pl.ds(k * have, have)], sem
                )
            )
            copies.append(
                pltpu.async_copy(
                    rscr.at[pl.ds(0, have)], rscr.at[pl.ds(k * have, have)], sem
                )
            )
        for c in copies:
            c.wait()
    rbs = [
        pltpu.async_copy(lscr.at[pl.ds(0, _RB)], b_sh.at[pl.ds(0, _RB)], sem),
        pltpu.async_copy(
            rscr.at[pl.ds(0, _RB)], b_sh.at[pl.ds(_LO + _T, _RB)], sem
        ),
        pltpu.async_copy(
            b_sh.at[pl.ds(_SEEDL, _FILL - _RB)], b_sh.at[pl.ds(_RB, _FILL - _RB)], sem
        ),
        pltpu.async_copy(
            b_sh.at[pl.ds(_SEEDR, _FILL - _RB)],
            b_sh.at[pl.ds(_LO + _T + _RB, _FILL - _RB)],
            sem,
        ),
    ]
    for c in rbs:
        c.wait()

    def issue(i, carry):
        row = r0 + i
        pltpu.async_copy(
            b_sh.at[pl.ds(_N - 1 - row, _N)], out_hbm.at[row], sem
        )
        return carry

    lax.fori_loop(0, _RPC, issue, 0)

    def drain(i, carry):
        pltpu.make_async_copy(
            out_hbm.at[0], b_sh.at[pl.ds(0, _N)], sem
        ).wait()
        return carry

    lax.fori_loop(0, _RPC, drain, 0)


def kernel(num_nodes, table):
    del num_nodes
    return _sc_band_fill(table)


# speedup vs baseline: 1.2787x; 1.2787x over previous
import functools

import jax
import jax.numpy as jnp
from jax import lax
from jax.experimental import pallas as pl
from jax.experimental.pallas import tpu as pltpu
from jax.experimental.pallas import tpu_sc as plsc

_N = 512
_D = 128
_K = 10
_T = 2 * _K + 1
_NC = 2
_RPC = _N // _NC
_LO = _N - 11
_SEED = 64
_RF = _N


@functools.partial(
    pl.kernel,
    out_type=jax.ShapeDtypeStruct((_N, _N, _D), jnp.float32),
    mesh=plsc.ScalarSubcoreMesh(axis_name="c", num_cores=_NC),
    scratch_types=[
        pltpu.VMEM_SHARED((2 * _N, _D), jnp.float32),
        pltpu.SemaphoreType.DMA,
    ],
)
def _sc_band_fill(table_hbm, out_hbm, b_sh, sem):
    cid = lax.axis_index("c")
    r0 = cid * _RPC
    lscr = out_hbm.at[r0]
    rscr = out_hbm.at[r0 + 1]
    pltpu.sync_copy(table_hbm, b_sh.at[pl.ds(_LO, _T)])
    cs = []
    for k in range(8):
        cs.append(
            pltpu.async_copy(b_sh.at[pl.ds(_LO, 1)], b_sh.at[pl.ds(k, 1)], sem)
        )
        cs.append(
            pltpu.async_copy(
                b_sh.at[pl.ds(_LO + _T - 1, 1)], b_sh.at[pl.ds(_RF + k, 1)], sem
            )
        )
    for c in cs:
        c.wait()
    cs = []
    for k in range(1, 8):
        cs.append(
            pltpu.async_copy(
                b_sh.at[pl.ds(0, 8)], b_sh.at[pl.ds(8 * k, 8)], sem
            )
        )
        cs.append(
            pltpu.async_copy(
                b_sh.at[pl.ds(_RF, 8)], b_sh.at[pl.ds(_RF + 8 * k, 8)], sem
            )
        )
    for c in cs:
        c.wait()
    s0 = pltpu.async_copy(b_sh.at[pl.ds(0, _SEED)], lscr.at[pl.ds(0, _SEED)], sem)
    s1 = pltpu.async_copy(
        b_sh.at[pl.ds(_RF, _SEED)], rscr.at[pl.ds(0, _SEED)], sem
    )
    s0.wait()
    s1.wait()
    cs = []
    for k in range(1, 8):
        cs.append(
            pltpu.async_copy(
                lscr.at[pl.ds(0, _SEED)], b_sh.at[pl.ds(_SEED * k, _SEED)], sem
            )
        )
        cs.append(
            pltpu.async_copy(
                rscr.at[pl.ds(0, _SEED)],
                b_sh.at[pl.ds(_RF + _SEED * k, _SEED)],
                sem,
            )
        )
    for c in cs:
        c.wait()
    pltpu.sync_copy(table_hbm, b_sh.at[pl.ds(_LO, _T)])

    def issue(i, carry):
        row = r0 + i
        pltpu.async_copy(
            b_sh.at[pl.ds(_N - 1 - row, _N)], out_hbm.at[row], sem
        )
        return carry

    lax.fori_loop(0, _RPC, issue, 0)

    def drain(i, carry):
        pltpu.make_async_copy(
            out_hbm.at[0], b_sh.at[pl.ds(0, _N)], sem
        ).wait()
        return carry

    lax.fori_loop(0, _RPC, drain, 0)


def kernel(num_nodes, table):
    del num_nodes
    return _sc_band_fill(table)


# speedup vs baseline: 1.2866x; 1.0061x over previous
import functools

import jax
import jax.numpy as jnp
from jax import lax
from jax.experimental import pallas as pl
from jax.experimental.pallas import tpu as pltpu
from jax.experimental.pallas import tpu_sc as plsc

_N = 512
_D = 128
_K = 10
_T = 2 * _K + 1
_NC = 2
_RPC = _N // _NC
_LO = _N - 11
_S = 64
_RF = _LO + _T
_EARLY = _K + 1


@functools.partial(
    pl.kernel,
    out_type=jax.ShapeDtypeStruct((_N, _N, _D), jnp.float32),
    mesh=plsc.ScalarSubcoreMesh(axis_name="c", num_cores=_NC),
    scratch_types=[
        pltpu.VMEM_SHARED((_RF + 8 * _S, _D), jnp.float32),
        pltpu.SemaphoreType.DMA,
    ],
)
def _sc_band_fill(table_hbm, out_hbm, b_sh, sem):
    cid = lax.axis_index("c")
    r0 = cid * _RPC
    lscr = out_hbm.at[r0]
    rscr = out_hbm.at[r0 + 1]
    pltpu.sync_copy(table_hbm, b_sh.at[pl.ds(_LO, _T)])
    cs = []
    for k in range(8):
        cs.append(
            pltpu.async_copy(b_sh.at[pl.ds(_LO, 1)], b_sh.at[pl.ds(k, 1)], sem)
        )
        cs.append(
            pltpu.async_copy(
                b_sh.at[pl.ds(_LO + _T - 1, 1)], b_sh.at[pl.ds(_RF + k, 1)], sem
            )
        )
    for c in cs:
        c.wait()
    cs = []
    for k in range(1, 8):
        cs.append(
            pltpu.async_copy(b_sh.at[pl.ds(0, 8)], b_sh.at[pl.ds(8 * k, 8)], sem)
        )
        cs.append(
            pltpu.async_copy(
                b_sh.at[pl.ds(_RF, 8)], b_sh.at[pl.ds(_RF + 8 * k, 8)], sem
            )
        )
    for c in cs:
        c.wait()
    s0 = pltpu.async_copy(b_sh.at[pl.ds(0, _S)], lscr.at[pl.ds(0, _S)], sem)
    s1 = pltpu.async_copy(b_sh.at[pl.ds(_RF, _S)], rscr.at[pl.ds(0, _S)], sem)
    s0.wait()
    s1.wait()

    def read_right_flank():
        return [
            pltpu.async_copy(
                rscr.at[pl.ds(0, _S)], b_sh.at[pl.ds(_RF + _S * k, _S)], sem
            )
            for k in range(1, 8)
        ]

    def read_left_flank():
        cs = [
            pltpu.async_copy(
                lscr.at[pl.ds(0, _S)], b_sh.at[pl.ds(_S * k, _S)], sem
            )
            for k in range(1, 7)
        ]
        cs.append(
            pltpu.async_copy(
                lscr.at[pl.ds(0, 48)], b_sh.at[pl.ds(6 * _S + _S, 48)], sem
            )
        )
        cs.append(
            pltpu.async_copy(b_sh.at[pl.ds(0, 5)], b_sh.at[pl.ds(496, 5)], sem)
        )
        return cs

    def issue_rows(lo, hi):
        def issue(i, carry):
            row = r0 + i
            pltpu.async_copy(
                b_sh.at[pl.ds(_N - 1 - row, _N)], out_hbm.at[row], sem
            )
            return carry

        lax.fori_loop(lo, hi, issue, 0)

    @pl.when(cid == 0)
    def _():
        for c in read_right_flank():
            c.wait()
        issue_rows(0, _EARLY)
        for c in read_left_flank():
            c.wait()
        issue_rows(_EARLY, _RPC)

    @pl.when(cid == 1)
    def _():
        for c in read_left_flank():
            c.wait()
        issue_rows(_RPC - _EARLY, _RPC)
        for c in read_right_flank():
            c.wait()
        issue_rows(0, _RPC - _EARLY)

    def drain(i, carry):
        pltpu.make_async_copy(
            out_hbm.at[0], b_sh.at[pl.ds(0, _N)], sem
        ).wait()
        return carry

    lax.fori_loop(0, _RPC, drain, 0)


def kernel(num_nodes, table):
    del num_nodes
    return _sc_band_fill(table)
